# baseline (speedup 1.0000x reference)
import functools

import jax
import jax.numpy as jnp
from jax import lax
from jax.experimental import pallas as pl
from jax.experimental.pallas import tpu as pltpu
from jax.experimental.pallas import tpu_sc as plsc

B = 16384
V = 1000000
D = 64
L = 16
NC = 2
NS = 16
NW = NC * NS
BPW = B // NW
CH = 128
NCH = BPW // CH
RB = 8000

_mesh = plsc.VectorSubcoreMesh(core_axis_name="c", subcore_axis_name="s")


def _repack_body(u_in, i_in, u_out, i_out):
    for src, dst in ((u_in, u_out), (i_in, i_out)):
        even = src[pl.Slice(0, RB // 2, 2), :]
        odd = src[pl.Slice(1, RB // 2, 2), :]
        dst[...] = jnp.concatenate([even, odd], axis=1)


_repack = pl.pallas_call(
    _repack_body,
    grid=(V // RB,),
    in_specs=[
        pl.BlockSpec((RB, D), lambda i: (i, 0)),
        pl.BlockSpec((RB, D), lambda i: (i, 0)),
    ],
    out_specs=[
        pl.BlockSpec((RB // 2, 2 * D), lambda i: (i, 0)),
        pl.BlockSpec((RB // 2, 2 * D), lambda i: (i, 0)),
    ],
    out_shape=[
        jax.ShapeDtypeStruct((V // 2, 2 * D), jnp.float32),
        jax.ShapeDtypeStruct((V // 2, 2 * D), jnp.float32),
    ],
)


@functools.partial(
    pl.kernel,
    out_type=jax.ShapeDtypeStruct((B,), jnp.float32),
    mesh=_mesh,
    compiler_params=pltpu.CompilerParams(use_tc_tiling_on_sc=True),
    scratch_types=[
        pltpu.VMEM((BPW,), jnp.int32),
        pltpu.VMEM((BPW,), jnp.int32),
        pltpu.VMEM((BPW,), jnp.int32),
        pltpu.VMEM((BPW,), jnp.int32),
        pltpu.VMEM((CH, 2 * D), jnp.float32),
        pltpu.VMEM((CH, 2 * D), jnp.float32),
        pltpu.VMEM((BPW,), jnp.float32),
        pltpu.SemaphoreType.DMA,
    ],
)
def _mf_sc(user_hbm, item_hbm, uemb_hbm, iemb_hbm, out_hbm,
           idx_u, idx_i, fat_u, fat_i, u_rows, i_rows, out_v, sem):
    wid = lax.axis_index("s") * NC + lax.axis_index("c")
    base = wid * BPW

    for c in range(NCH):
        pltpu.sync_copy(user_hbm.at[pl.ds(base + c * CH, CH)],
                        idx_u.at[pl.ds(c * CH, CH)])
        pltpu.sync_copy(item_hbm.at[pl.ds(base + c * CH, CH)],
                        idx_i.at[pl.ds(c * CH, CH)])

    def prep(k, carry):
        fat_u[pl.ds(k * L, L)] = lax.shift_right_logical(idx_u[pl.ds(k * L, L)], 1)
        fat_i[pl.ds(k * L, L)] = lax.shift_right_logical(idx_i[pl.ds(k * L, L)], 1)
        return carry

    lax.fori_loop(0, BPW // L, prep, 0)

    lane = lax.iota(jnp.int32, L)
    bitrev = (((lane & 1) << 3) | ((lane & 2) << 1)
              | ((lane & 4) >> 1) | ((lane & 8) >> 3))

    def permute(x, idx):
        return lax.gather(
            x, idx[:, None],
            dimension_numbers=lax.GatherDimensionNumbers(
                offset_dims=(), collapsed_slice_dims=(0,),
                start_index_map=(0,)),
            slice_sizes=(1,),
            mode=lax.GatherScatterMode.PROMISE_IN_BOUNDS)

    def group_body(c, g, _):
        r0 = g * L
        pu = idx_u[pl.ds(c * CH + r0, L)] & 1
        pi = idx_i[pl.ds(c * CH + r0, L)] & 1
        ps = []
        for k in range(L):
            acc = None
            for m in range(D // L):
                uu = jnp.where(pu[k] == 0,
                               u_rows[r0 + k, pl.ds(m * L, L)],
                               u_rows[r0 + k, pl.ds(D + m * L, L)])
                ii = jnp.where(pi[k] == 0,
                               i_rows[r0 + k, pl.ds(m * L, L)],
                               i_rows[r0 + k, pl.ds(D + m * L, L)])
                p = uu * ii
                acc = p if acc is None else acc + p
            ps.append(acc)
        d = L // 2
        while len(ps) > 1:
            sel = (lane & d) == 0
            nxt = []
            for m in range(0, len(ps), 2):
                fa = ps[m] + permute(ps[m], lane ^ d)
                fb = ps[m + 1] + permute(ps[m + 1], lane ^ d)
                nxt.append(jnp.where(sel, fa, fb))
            ps = nxt
            d //= 2
        out_v[pl.ds(c * CH + r0, L)] = permute(ps[0], bitrev)
        return _

    def chunk_body(c, carry):
        sl = pl.ds(c * CH, CH)
        cu = pltpu.async_copy(uemb_hbm.at[fat_u.at[sl]], u_rows, sem)
        ci = pltpu.async_copy(iemb_hbm.at[fat_i.at[sl]], i_rows, sem)
        cu.wait()
        ci.wait()
        lax.fori_loop(0, CH // L, functools.partial(group_body, c), 0)
        return carry

    lax.fori_loop(0, NCH, chunk_body, 0)

    pltpu.sync_copy(out_v, out_hbm.at[pl.ds(base, BPW)])


def kernel(user, item, user_emb, item_emb):
    u2, i2 = _repack(user_emb, item_emb)
    return _mf_sc(user, item, u2, i2)


# speedup vs baseline: 1.0445x; 1.0445x over previous
import functools

import jax
import jax.numpy as jnp
from jax import lax
from jax.experimental import pallas as pl
from jax.experimental.pallas import tpu as pltpu
from jax.experimental.pallas import tpu_sc as plsc

B = 16384
V = 1000000
D = 64
L = 16
NC = 2
NS = 16
NW = NC * NS
BPW = B // NW
CH = 128
NCH = BPW // CH
RB = 8000

_mesh = plsc.VectorSubcoreMesh(core_axis_name="c", subcore_axis_name="s")


def _repack_body(u_in, i_in, u_out, i_out):
    for src, dst in ((u_in, u_out), (i_in, i_out)):
        even = src[pl.Slice(0, RB // 2, 2), :]
        odd = src[pl.Slice(1, RB // 2, 2), :]
        dst[...] = jnp.concatenate([even, odd], axis=1)


_repack = pl.pallas_call(
    _repack_body,
    grid=(V // RB,),
    in_specs=[
        pl.BlockSpec((RB, D), lambda i: (i, 0)),
        pl.BlockSpec((RB, D), lambda i: (i, 0)),
    ],
    out_specs=[
        pl.BlockSpec((RB // 2, 2 * D), lambda i: (i, 0)),
        pl.BlockSpec((RB // 2, 2 * D), lambda i: (i, 0)),
    ],
    out_shape=[
        jax.ShapeDtypeStruct((V // 2, 2 * D), jnp.float32),
        jax.ShapeDtypeStruct((V // 2, 2 * D), jnp.float32),
    ],
)


@functools.partial(
    pl.kernel,
    out_type=jax.ShapeDtypeStruct((B,), jnp.float32),
    mesh=_mesh,
    compiler_params=pltpu.CompilerParams(use_tc_tiling_on_sc=True),
    scratch_types=[
        pltpu.VMEM((BPW,), jnp.int32),
        pltpu.VMEM((BPW,), jnp.int32),
        pltpu.VMEM((BPW,), jnp.int32),
        pltpu.VMEM((BPW,), jnp.int32),
        pltpu.VMEM((CH, 2 * D), jnp.float32),
        pltpu.VMEM((CH, 2 * D), jnp.float32),
        pltpu.VMEM((BPW,), jnp.float32),
        pltpu.SemaphoreType.DMA,
    ],
)
def _mf_sc(user_hbm, item_hbm, uemb_hbm, iemb_hbm, out_hbm,
           idx_u, idx_i, fat_u, fat_i, u_rows, i_rows, out_v, sem):
    wid = lax.axis_index("s") * NC + lax.axis_index("c")
    base = wid * BPW

    for c in range(NCH):
        pltpu.sync_copy(user_hbm.at[pl.ds(base + c * CH, CH)],
                        idx_u.at[pl.ds(c * CH, CH)])
        pltpu.sync_copy(item_hbm.at[pl.ds(base + c * CH, CH)],
                        idx_i.at[pl.ds(c * CH, CH)])

    def prep(k, carry):
        fat_u[pl.ds(k * L, L)] = lax.shift_right_logical(idx_u[pl.ds(k * L, L)], 1)
        fat_i[pl.ds(k * L, L)] = lax.shift_right_logical(idx_i[pl.ds(k * L, L)], 1)
        return carry

    lax.fori_loop(0, BPW // L, prep, 0)

    lane = lax.iota(jnp.int32, L)
    bitrev = (((lane & 1) << 3) | ((lane & 2) << 1)
              | ((lane & 4) >> 1) | ((lane & 8) >> 3))

    def permute(x, idx):
        return lax.gather(
            x, idx[:, None],
            dimension_numbers=lax.GatherDimensionNumbers(
                offset_dims=(), collapsed_slice_dims=(0,),
                start_index_map=(0,)),
            slice_sizes=(1,),
            mode=lax.GatherScatterMode.PROMISE_IN_BOUNDS)

    def group_body(c, g, _):
        r0 = g * L
        pu = idx_u[pl.ds(c * CH + r0, L)] & 1
        pi = idx_i[pl.ds(c * CH + r0, L)] & 1
        ps = []
        for k in range(L):
            acc = None
            for m in range(D // L):
                uu = jnp.where(pu[k] == 0,
                               u_rows[r0 + k, pl.ds(m * L, L)],
                               u_rows[r0 + k, pl.ds(D + m * L, L)])
                ii = jnp.where(pi[k] == 0,
                               i_rows[r0 + k, pl.ds(m * L, L)],
                               i_rows[r0 + k, pl.ds(D + m * L, L)])
                p = uu * ii
                acc = p if acc is None else acc + p
            ps.append(acc)
        d = L // 2
        while len(ps) > 1:
            sel = (lane & d) == 0
            nxt = []
            for m in range(0, len(ps), 2):
                fa = ps[m] + permute(ps[m], lane ^ d)
                fb = ps[m + 1] + permute(ps[m + 1], lane ^ d)
                nxt.append(jnp.where(sel, fa, fb))
            ps = nxt
            d //= 2
        out_v[pl.ds(c * CH + r0, L)] = permute(ps[0], bitrev)
        return _

    def chunk_body(c, carry):
        sl = pl.ds(c * CH, CH)
        cu = pltpu.async_copy(uemb_hbm.at[fat_u.at[sl]], u_rows, sem)
        ci = pltpu.async_copy(iemb_hbm.at[fat_i.at[sl]], i_rows, sem)
        cu.wait()
        ci.wait()
        lax.fori_loop(0, CH // L, functools.partial(group_body, c), 0)
        return carry

    lax.fori_loop(0, NCH, chunk_body, 0)

    pltpu.sync_copy(out_v, out_hbm.at[pl.ds(base, BPW)])


def kernel(user, item, user_emb, item_emb):
    u2 = user_emb.reshape(V // 2, 2 * D)
    i2 = item_emb.reshape(V // 2, 2 * D)
    return _mf_sc(user, item, u2, i2)


# speedup vs baseline: 1.6485x; 1.5783x over previous
import functools

import jax
import jax.numpy as jnp
from jax import lax
from jax.experimental import pallas as pl
from jax.experimental.pallas import tpu as pltpu
from jax.experimental.pallas import tpu_sc as plsc

B = 16384
D = 64
L = 16
NC = 2
NS = 16
NW = NC * NS
BPW = B // NW
G = 16
NCH = BPW // G

_mesh = plsc.VectorSubcoreMesh(core_axis_name="c", subcore_axis_name="s")


@functools.partial(
    pl.kernel,
    out_type=jax.ShapeDtypeStruct((B,), jnp.float32),
    mesh=_mesh,
    compiler_params=pltpu.CompilerParams(use_tc_tiling_on_sc=True),
    scratch_types=[
        pltpu.VMEM((BPW,), jnp.int32),
        pltpu.VMEM((BPW,), jnp.int32),
        pltpu.VMEM((2, G, D), jnp.float32),
        pltpu.VMEM((2, G, D), jnp.float32),
        pltpu.VMEM((BPW,), jnp.float32),
        pltpu.SemaphoreType.DMA,
        pltpu.SemaphoreType.DMA,
    ],
)
def _mf_sc(user_hbm, item_hbm, uemb_hbm, iemb_hbm, out_hbm,
           idx_u, idx_i, urows, irows, out_v, sem0, sem1):
    wid = lax.axis_index("s") * NC + lax.axis_index("c")
    base = wid * BPW
    sems = (sem0, sem1)

    for c4 in range(BPW // 128):
        pltpu.sync_copy(user_hbm.at[pl.ds(base + c4 * 128, 128)],
                        idx_u.at[pl.ds(c4 * 128, 128)])
        pltpu.sync_copy(item_hbm.at[pl.ds(base + c4 * 128, 128)],
                        idx_i.at[pl.ds(c4 * 128, 128)])

    def fire(c, slot):
        vu = idx_u[pl.ds(c * G, L)]
        vi = idx_i[pl.ds(c * G, L)]
        for j in range(G):
            pltpu.async_copy(uemb_hbm.at[vu[j]], urows.at[slot, j], sems[slot])
            pltpu.async_copy(iemb_hbm.at[vi[j]], irows.at[slot, j], sems[slot])

    def drain(slot):
        for j in range(G):
            pltpu.make_async_copy(uemb_hbm.at[0], urows.at[slot, j],
                                  sems[slot]).wait()
            pltpu.make_async_copy(iemb_hbm.at[0], irows.at[slot, j],
                                  sems[slot]).wait()

    lane = lax.iota(jnp.int32, L)
    bitrev = (((lane & 1) << 3) | ((lane & 2) << 1)
              | ((lane & 4) >> 1) | ((lane & 8) >> 3))

    def permute(x, idx):
        return lax.gather(
            x, idx[:, None],
            dimension_numbers=lax.GatherDimensionNumbers(
                offset_dims=(), collapsed_slice_dims=(0,),
                start_index_map=(0,)),
            slice_sizes=(1,),
            mode=lax.GatherScatterMode.PROMISE_IN_BOUNDS)

    def compute(c, slot):
        ps = []
        for j in range(G):
            acc = (urows[slot, j, pl.ds(0, L)]
                   * irows[slot, j, pl.ds(0, L)])
            for m in range(1, D // L):
                acc = acc + (urows[slot, j, pl.ds(m * L, L)]
                             * irows[slot, j, pl.ds(m * L, L)])
            ps.append(acc)
        d = L // 2
        while len(ps) > 1:
            sel = (lane & d) == 0
            nxt = []
            for m in range(0, len(ps), 2):
                fa = ps[m] + permute(ps[m], lane ^ d)
                fb = ps[m + 1] + permute(ps[m + 1], lane ^ d)
                nxt.append(jnp.where(sel, fa, fb))
            ps = nxt
            d //= 2
        out_v[pl.ds(c * G, G)] = permute(ps[0], bitrev)

    fire(0, 0)

    def pair_body(t, carry):
        c0 = t * 2
        fire(c0 + 1, 1)
        drain(0)
        compute(c0, 0)

        @pl.when(c0 + 2 < NCH)
        def _():
            fire(c0 + 2, 0)

        drain(1)
        compute(c0 + 1, 1)
        return carry

    lax.fori_loop(0, NCH // 2, pair_body, 0)

    pltpu.sync_copy(out_v, out_hbm.at[pl.ds(base, BPW)])


def kernel(user, item, user_emb, item_emb):
    return _mf_sc(user, item, user_emb, item_emb)
